# baseline (speedup 1.0000x reference)
import jax
import jax.numpy as jnp
from jax import lax
from jax.experimental import pallas as pl
from jax.experimental.pallas import tpu as pltpu
from jax.experimental.pallas import tpu_sc as plsc

_EMB = 128
_NFEAT = 7
_N = 100000
_FLAT = _N * _NFEAT
_NC, _NS = 2, 16
_NW = _NC * _NS
_CHUNK = 128
_NCHUNK = 171
_PER_W = _NCHUNK * _CHUNK
_FLAT_PAD = _NW * _PER_W
_NGRP = _PER_W // 16
_TOTAL_ROWS = 590
_OFFS = (0, 4, 261, 269, 326, 329, 332)


def _sc_body(tab_hbm, x_hbm, out_hbm,
             xraw_v, idx_v, rows_a, rows_b, sem_a, sem_b):
    wid = lax.axis_index("s") * _NC + lax.axis_index("c")
    pltpu.sync_copy(x_hbm.at[wid], xraw_v)

    lane = lax.iota(jnp.int32, 16)
    base = wid * _PER_W

    def idx_body(t, carry):
        q = t * 16
        x16 = xraw_v[pl.ds(q, 16)]
        ph = lax.rem(base + q + lane, _NFEAT)
        off16 = jnp.int32(_OFFS[_NFEAT - 1])
        for f in range(_NFEAT - 2, -1, -1):
            off16 = jnp.where(ph == f, jnp.int32(_OFFS[f]), off16)
        idx_v[pl.ds(q, 16)] = jnp.clip(x16 + off16, 0, _TOTAL_ROWS - 1)
        return carry

    lax.fori_loop(0, _NGRP, idx_body, None)

    def g_desc(c, rows, sem):
        return pltpu.make_async_copy(
            tab_hbm.at[idx_v.at[pl.ds(c * _CHUNK, _CHUNK)]], rows, sem)

    out_base = wid * _PER_W

    g_desc(0, rows_a, sem_a).start()
    g_desc(1, rows_b, sem_b).start()

    def pipe(i, carry):
        ca = 2 * i
        g_desc(ca, rows_a, sem_a).wait()
        pltpu.sync_copy(rows_a, out_hbm.at[pl.ds(out_base + ca * _CHUNK, _CHUNK)])
        g_desc(ca + 2, rows_a, sem_a).start()
        cb = ca + 1
        g_desc(cb, rows_b, sem_b).wait()
        pltpu.sync_copy(rows_b, out_hbm.at[pl.ds(out_base + cb * _CHUNK, _CHUNK)])

        @pl.when(i < _NCHUNK // 2 - 1)
        def _():
            g_desc(cb + 2, rows_b, sem_b).start()

        return carry

    lax.fori_loop(0, _NCHUNK // 2, pipe, None)

    last = _NCHUNK - 1
    g_desc(last, rows_a, sem_a).wait()
    pltpu.sync_copy(rows_a, out_hbm.at[pl.ds(out_base + last * _CHUNK, _CHUNK)])


def kernel(x, W0, W1, W2, W3, W4, W5, W6):
    tab = jnp.concatenate([W0, W1, W2, W3, W4, W5, W6], axis=0)
    xf = jnp.pad(x.reshape(-1), (0, _FLAT_PAD - _FLAT)).reshape(_NW, _PER_W)
    run = pl.kernel(
        _sc_body,
        out_type=jax.ShapeDtypeStruct((_FLAT_PAD, _EMB), jnp.float32),
        mesh=plsc.VectorSubcoreMesh(core_axis_name="c", subcore_axis_name="s"),
        scratch_types=[
            pltpu.VMEM((_PER_W,), jnp.int32),
            pltpu.VMEM((_PER_W,), jnp.int32),
            pltpu.VMEM((_CHUNK, _EMB), jnp.float32),
            pltpu.VMEM((_CHUNK, _EMB), jnp.float32),
            pltpu.SemaphoreType.DMA,
            pltpu.SemaphoreType.DMA,
        ],
    )
    out = run(tab, xf)
    return out[:_FLAT].reshape(_N, _NFEAT * _EMB)


# speedup vs baseline: 4.5496x; 4.5496x over previous
import jax
import jax.numpy as jnp
from jax import lax
from jax.experimental import pallas as pl
from jax.experimental.pallas import tpu as pltpu
from jax.experimental.pallas import tpu_sc as plsc

_EMB = 128
_NFEAT = 7
_N = 100000
_FLAT = _N * _NFEAT
_NC, _NS = 2, 16
_NW = _NC * _NS
_CHUNK = 128
_NCHUNK = 171
_PER_W = _NCHUNK * _CHUNK
_FLAT_PAD = _NW * _PER_W
_NGRP = _PER_W // 16
_TOTAL_ROWS = 590
_OFFS = (0, 4, 261, 269, 326, 329, 332)


def _sc_body(tab_hbm, x_hbm, out_hbm,
             tab_sp, xraw_v, idx_v, rows_a, rows_b, sem_a, sem_b):
    sid = lax.axis_index("s")
    wid = sid * _NC + lax.axis_index("c")

    @pl.when(sid == 0)
    def _():
        pltpu.sync_copy(tab_hbm, tab_sp)

    pltpu.sync_copy(x_hbm.at[wid], xraw_v)
    plsc.subcore_barrier()

    lane = lax.iota(jnp.int32, 16)
    base = wid * _PER_W

    def idx_body(t, carry):
        q = t * 16
        x16 = xraw_v[pl.ds(q, 16)]
        ph = lax.rem(base + q + lane, _NFEAT)
        off16 = jnp.int32(_OFFS[_NFEAT - 1])
        for f in range(_NFEAT - 2, -1, -1):
            off16 = jnp.where(ph == f, jnp.int32(_OFFS[f]), off16)
        idx_v[pl.ds(q, 16)] = jnp.clip(x16 + off16, 0, _TOTAL_ROWS - 1)
        return carry

    lax.fori_loop(0, _NGRP, idx_body, None)

    def g_desc(c, rows, sem):
        return pltpu.make_async_copy(
            tab_sp.at[idx_v.at[pl.ds(c * _CHUNK, _CHUNK)]], rows, sem)

    out_base = wid * _PER_W

    g_desc(0, rows_a, sem_a).start()
    g_desc(1, rows_b, sem_b).start()

    def pipe(i, carry):
        ca = 2 * i
        g_desc(ca, rows_a, sem_a).wait()
        pltpu.sync_copy(rows_a, out_hbm.at[pl.ds(out_base + ca * _CHUNK, _CHUNK)])
        g_desc(ca + 2, rows_a, sem_a).start()
        cb = ca + 1
        g_desc(cb, rows_b, sem_b).wait()
        pltpu.sync_copy(rows_b, out_hbm.at[pl.ds(out_base + cb * _CHUNK, _CHUNK)])

        @pl.when(i < _NCHUNK // 2 - 1)
        def _():
            g_desc(cb + 2, rows_b, sem_b).start()

        return carry

    lax.fori_loop(0, _NCHUNK // 2, pipe, None)

    last = _NCHUNK - 1
    g_desc(last, rows_a, sem_a).wait()
    pltpu.sync_copy(rows_a, out_hbm.at[pl.ds(out_base + last * _CHUNK, _CHUNK)])


def kernel(x, W0, W1, W2, W3, W4, W5, W6):
    tab = jnp.concatenate([W0, W1, W2, W3, W4, W5, W6], axis=0)
    xf = jnp.pad(x.reshape(-1), (0, _FLAT_PAD - _FLAT)).reshape(_NW, _PER_W)
    run = pl.kernel(
        _sc_body,
        out_type=jax.ShapeDtypeStruct((_FLAT_PAD, _EMB), jnp.float32),
        mesh=plsc.VectorSubcoreMesh(core_axis_name="c", subcore_axis_name="s"),
        scratch_types=[
            pltpu.VMEM_SHARED((_TOTAL_ROWS, _EMB), jnp.float32),
            pltpu.VMEM((_PER_W,), jnp.int32),
            pltpu.VMEM((_PER_W,), jnp.int32),
            pltpu.VMEM((_CHUNK, _EMB), jnp.float32),
            pltpu.VMEM((_CHUNK, _EMB), jnp.float32),
            pltpu.SemaphoreType.DMA,
            pltpu.SemaphoreType.DMA,
        ],
    )
    out = run(tab, xf)
    return out[:_FLAT].reshape(_N, _NFEAT * _EMB)


# speedup vs baseline: 15.3567x; 3.3754x over previous
import jax
import jax.numpy as jnp
from jax import lax
from jax.experimental import pallas as pl
from jax.experimental.pallas import tpu as pltpu
from jax.experimental.pallas import tpu_sc as plsc

_EMB = 128
_NFEAT = 7
_N = 100000
_OUT_D = _NFEAT * _EMB
_NC, _NS = 2, 16
_NACT = 25
_NODES_W = _N // _NACT
_CNODES = 128
_NFULL = _NODES_W // _CNODES
_TAIL = _NODES_W - _NFULL * _CNODES
_XSTRIDE = 4096
_TOTAL_ROWS = 590
_OFFS = (0, 4, 261, 269, 326, 329, 332)


def _sc_body(tab_hbm, xt_hbm, out_hbm,
             tab_sp, xrows_v, idx_v, rows_a, rows_b, sem_a, sem_b):
    sid = lax.axis_index("s")
    wid = sid * _NC + lax.axis_index("c")

    @pl.when(sid == 0)
    def _():
        pltpu.sync_copy(tab_hbm, tab_sp)

    plsc.subcore_barrier()

    @pl.when(wid < _NACT)
    def _():
        node0 = wid * _NODES_W
        for k in range(_NFEAT):
            pltpu.sync_copy(xt_hbm.at[pl.ds(k * _N + node0, _NODES_W)],
                            xrows_v.at[pl.ds(k * _XSTRIDE, _NODES_W)])

        bufs = (rows_a, rows_b)
        sems = (sem_a, sem_b)

        def build_idx(c):
            for k in range(_NFEAT):
                for h in range(8):
                    x16 = xrows_v[pl.ds(k * _XSTRIDE + c * _CNODES + h * 16, 16)]
                    idx_v[pl.ds(k * _EMB + h * 16, 16)] = jnp.clip(
                        x16 + jnp.int32(_OFFS[k]), 0, _TOTAL_ROWS - 1)

        def g_desc(k, b):
            return pltpu.make_async_copy(
                tab_sp.at[idx_v.at[pl.ds(k * _EMB, _EMB)]], bufs[b], sems[b])

        def do_chunk(c, n_out):
            build_idx(c)
            g_desc(0, 0).start()
            for k in range(_NFEAT):
                b = k % 2
                if k + 1 < _NFEAT:
                    g_desc(k + 1, 1 - b).start()
                g_desc(k, b).wait()
                pltpu.sync_copy(
                    bufs[b].at[pl.ds(0, n_out)],
                    out_hbm.at[pl.ds(node0 + c * _CNODES, n_out),
                               pl.ds(k * _EMB, _EMB)])

        lax.fori_loop(0, _NFULL, lambda c, car: (do_chunk(c, _CNODES), car)[1],
                      None)
        do_chunk(_NFULL, _TAIL)


def kernel(x, W0, W1, W2, W3, W4, W5, W6):
    tab = jnp.concatenate([W0, W1, W2, W3, W4, W5, W6], axis=0)
    xt = x.T.reshape(-1)
    run = pl.kernel(
        _sc_body,
        out_type=jax.ShapeDtypeStruct((_N, _OUT_D), jnp.float32),
        mesh=plsc.VectorSubcoreMesh(core_axis_name="c", subcore_axis_name="s"),
        scratch_types=[
            pltpu.VMEM_SHARED((_TOTAL_ROWS, _EMB), jnp.float32),
            pltpu.VMEM((_NFEAT * _XSTRIDE,), jnp.int32),
            pltpu.VMEM((_NFEAT * _EMB,), jnp.int32),
            pltpu.VMEM((_CNODES, _EMB), jnp.float32),
            pltpu.VMEM((_CNODES, _EMB), jnp.float32),
            pltpu.SemaphoreType.DMA,
            pltpu.SemaphoreType.DMA,
        ],
    )
    return run(tab, xt)


# speedup vs baseline: 19.4287x; 1.2652x over previous
import jax
import jax.numpy as jnp
from jax import lax
from jax.experimental import pallas as pl
from jax.experimental.pallas import tpu as pltpu
from jax.experimental.pallas import tpu_sc as plsc

_EMB = 128
_NFEAT = 7
_N = 100000
_OUT_D = _NFEAT * _EMB
_NC, _NS = 2, 16
_NW = _NC * _NS
_BIGW = 20
_NPW = 3128
_CNODES = 128
_NFULL = 24
_NCH = _NFULL + 1
_XSTRIDE = _NCH * _CNODES
_NUNIT = _NFULL * _NFEAT
_XTPAD = _NW * 8
_TOTAL_ROWS = 590
_OFFS = (0, 4, 261, 269, 326, 329, 332)


def _sc_body(tab_hbm, xt_hbm, out_hbm,
             tab_sp, xrows_v, idx_v, rows_a, rows_b, sem_a, sem_b):
    sid = lax.axis_index("s")
    wid = sid * _NC + lax.axis_index("c")

    @pl.when(sid == 0)
    def _():
        pltpu.sync_copy(tab_hbm, tab_sp)

    plsc.subcore_barrier()

    node0 = wid * _NPW - 8 * jnp.maximum(wid - _BIGW, 0)
    tail = jnp.where(wid < _BIGW, _NPW - _NFULL * _CNODES,
                     _NPW - 8 - _NFULL * _CNODES)

    for k in range(_NFEAT):
        pltpu.sync_copy(xt_hbm.at[pl.ds(k * _N + node0, _NPW)],
                        xrows_v.at[pl.ds(k * _XSTRIDE, _NPW)])

    def build_idx(c, carry):
        for k in range(_NFEAT):
            for h in range(8):
                x16 = xrows_v[pl.ds(k * _XSTRIDE + c * _CNODES + h * 16, 16)]
                idx_v[pl.ds((c * _NFEAT + k) * _EMB + h * 16, 16)] = jnp.clip(
                    x16 + jnp.int32(_OFFS[k]), 0, _TOTAL_ROWS - 1)
        return carry

    lax.fori_loop(0, _NCH, build_idx, None)

    bufs = (rows_a, rows_b)
    sems = (sem_a, sem_b)

    def g_desc(u, b):
        return pltpu.make_async_copy(
            tab_sp.at[idx_v.at[pl.ds(u * _EMB, _EMB)]], bufs[b], sems[b])

    def out_stream(u, b, n_out):
        c = lax.div(u, _NFEAT)
        k = lax.rem(u, _NFEAT)
        pltpu.sync_copy(
            bufs[b].at[pl.ds(0, n_out)],
            out_hbm.at[pl.ds(node0 + c * _CNODES, n_out),
                       pl.ds(k * _EMB, _EMB)])

    g_desc(0, 0).start()
    g_desc(1, 1).start()

    def pipe(i, carry):
        ua = 2 * i
        g_desc(ua, 0).wait()
        out_stream(ua, 0, _CNODES)

        @pl.when(ua + 2 < _NUNIT)
        def _():
            g_desc(ua + 2, 0).start()

        ub = ua + 1
        g_desc(ub, 1).wait()
        out_stream(ub, 1, _CNODES)

        @pl.when(ub + 2 < _NUNIT)
        def _():
            g_desc(ub + 2, 1).start()

        return carry

    lax.fori_loop(0, _NUNIT // 2, pipe, None)

    def do_tail(n_out):
        g_desc(_NUNIT, 0).start()
        for k in range(_NFEAT):
            b = k % 2
            if k + 1 < _NFEAT:
                g_desc(_NUNIT + k + 1, 1 - b).start()
            g_desc(_NUNIT + k, b).wait()
            out_stream(_NUNIT + k, b, n_out)

    @pl.when(tail == 56)
    def _():
        do_tail(56)

    @pl.when(tail == 48)
    def _():
        do_tail(48)


def kernel(x, W0, W1, W2, W3, W4, W5, W6):
    tab = jnp.concatenate([W0, W1, W2, W3, W4, W5, W6], axis=0)
    xt = jnp.pad(x.T.reshape(-1), (0, _XTPAD))
    run = pl.kernel(
        _sc_body,
        out_type=jax.ShapeDtypeStruct((_N, _OUT_D), jnp.float32),
        mesh=plsc.VectorSubcoreMesh(core_axis_name="c", subcore_axis_name="s"),
        scratch_types=[
            pltpu.VMEM_SHARED((_TOTAL_ROWS, _EMB), jnp.float32),
            pltpu.VMEM((_NFEAT * _XSTRIDE,), jnp.int32),
            pltpu.VMEM((_NCH * _NFEAT * _EMB,), jnp.int32),
            pltpu.VMEM((_CNODES, _EMB), jnp.float32),
            pltpu.VMEM((_CNODES, _EMB), jnp.float32),
            pltpu.SemaphoreType.DMA,
            pltpu.SemaphoreType.DMA,
        ],
    )
    return run(tab, xt)
